# Initial kernel scaffold; baseline (speedup 1.0000x reference)
import functools

import jax
import jax.numpy as jnp
from jax import lax
from jax.experimental import pallas as pl
from jax.experimental.pallas import tpu as pltpu
from jax.experimental.pallas import tpu_sc as plsc

B = 16384
TAU = 5
_NUM_CORES = 2
_NUM_SUBCORES = 16
_NW = _NUM_CORES * _NUM_SUBCORES
_BPW = B // _NW
_LANES = 16

_mesh = plsc.VectorSubcoreMesh(core_axis_name="c", subcore_axis_name="s")


@functools.partial(
    pl.kernel,
    mesh=_mesh,
    out_type=jax.ShapeDtypeStruct((2 * B,), jnp.float32),
    scratch_types=[
        pltpu.VMEM((_BPW,), jnp.int32),
        pltpu.VMEM((2 * _BPW,), jnp.float32),
    ],
)
def _router(it_hbm, out_hbm, it_v, out_v):
    wid = lax.axis_index("s") * _NUM_CORES + lax.axis_index("c")
    base = wid * _BPW
    pltpu.sync_copy(it_hbm.at[pl.ds(base, _BPW)], it_v)

    lane = lax.iota(jnp.int32, _LANES)
    parity = lane & 1
    half = lane >> 1

    for k in range(2 * _BPW // _LANES):
        src = plsc.load_gather(it_v, [half + (_LANES // 2) * k])
        pred = (jnp.remainder(src + 1, TAU) == 0).astype(jnp.int32)
        out_v[pl.ds(_LANES * k, _LANES)] = (pred == parity).astype(jnp.float32)

    pltpu.sync_copy(out_v, out_hbm.at[pl.ds(2 * base, 2 * _BPW)])


def kernel(iteration):
    out_flat = _router(iteration.astype(jnp.int32))
    return out_flat.reshape(B, 2)


# speedup vs baseline: 1.6216x; 1.6216x over previous
import functools

import jax
import jax.numpy as jnp
from jax import lax
from jax.experimental import pallas as pl
from jax.experimental.pallas import tpu as pltpu
from jax.experimental.pallas import tpu_sc as plsc

B = 16384
TAU = 5
_NUM_CORES = 2
_NUM_SUBCORES = 16
_NW = _NUM_CORES * _NUM_SUBCORES
_BPW = B // _NW
_LANES = 16

_mesh = plsc.VectorSubcoreMesh(core_axis_name="c", subcore_axis_name="s")


@functools.partial(
    pl.kernel,
    mesh=_mesh,
    out_type=jax.ShapeDtypeStruct((2 * B,), jnp.float32),
    scratch_types=[
        pltpu.VMEM((_BPW,), jnp.int32),
        pltpu.VMEM((2 * _BPW,), jnp.float32),
    ],
    compiler_params=pltpu.CompilerParams(needs_layout_passes=False),
)
def _router(it_hbm, out_hbm, it_v, out_v):
    wid = lax.axis_index("s") * _NUM_CORES + lax.axis_index("c")
    base = wid * _BPW
    pltpu.sync_copy(it_hbm.at[pl.ds(base, _BPW)], it_v)

    lane = lax.iota(jnp.int32, _LANES)
    parity = lane & 1
    half = lane >> 1

    for k in range(2 * _BPW // _LANES):
        src = plsc.load_gather(it_v, [half + (_LANES // 2) * k])
        n = src + 1
        q = (n.astype(jnp.float32) * (1.0 / TAU) + 0.5).astype(jnp.int32)
        pred = (q * TAU == n).astype(jnp.int32)
        out_v[pl.ds(_LANES * k, _LANES)] = (pred == parity).astype(jnp.float32)

    pltpu.sync_copy(out_v, out_hbm.at[pl.ds(2 * base, 2 * _BPW)])


def kernel(iteration):
    out_flat = _router(iteration.astype(jnp.int32))
    return out_flat.reshape(B, 2)


# speedup vs baseline: 1.9905x; 1.2275x over previous
import functools

import jax
import jax.numpy as jnp
from jax import lax
from jax.experimental import pallas as pl
from jax.experimental.pallas import tpu as pltpu
from jax.experimental.pallas import tpu_sc as plsc

B = 16384
TAU = 5
_NUM_CORES = 2
_NUM_SUBCORES = 16
_NW = _NUM_CORES * _NUM_SUBCORES
_BPW = B // _NW
_LANES = 16

_mesh = plsc.VectorSubcoreMesh(core_axis_name="c", subcore_axis_name="s")


@functools.partial(
    pl.kernel,
    mesh=_mesh,
    out_type=jax.ShapeDtypeStruct((B, 2), jnp.float32),
    scratch_types=[
        pltpu.VMEM((_BPW,), jnp.int32),
        pltpu.VMEM((_BPW, 2), jnp.float32),
    ],
    compiler_params=pltpu.CompilerParams(needs_layout_passes=False),
)
def _router(it_hbm, out_hbm, it_v, out_v):
    wid = lax.axis_index("s") * _NUM_CORES + lax.axis_index("c")
    base = wid * _BPW
    pltpu.sync_copy(it_hbm.at[pl.ds(base, _BPW)], it_v)

    lane = lax.iota(jnp.int32, _LANES)
    parity = lane & 1
    half = lane >> 1

    for k in range(2 * _BPW // _LANES):
        row = half + (_LANES // 2) * k
        src = plsc.load_gather(it_v, [row])
        n = src + 1
        q = (n.astype(jnp.float32) * (1.0 / TAU) + 0.5).astype(jnp.int32)
        pred = (q * TAU == n).astype(jnp.int32)
        plsc.store_scatter(out_v, [row, parity], (pred == parity).astype(jnp.float32))

    pltpu.sync_copy(out_v, out_hbm.at[pl.ds(base, _BPW), :])


def kernel(iteration):
    return _router(iteration.astype(jnp.int32))


# speedup vs baseline: 1.9924x; 1.0010x over previous
import functools

import jax
import jax.numpy as jnp
from jax import lax
from jax.experimental import pallas as pl
from jax.experimental.pallas import tpu as pltpu
from jax.experimental.pallas import tpu_sc as plsc

B = 16384
TAU = 5
_NUM_CORES = 2
_NUM_SUBCORES = 16
_NW = _NUM_CORES * _NUM_SUBCORES
_BPW = B // _NW
_LANES = 16

_mesh = plsc.VectorSubcoreMesh(core_axis_name="c", subcore_axis_name="s")


@functools.partial(
    pl.kernel,
    mesh=_mesh,
    out_type=jax.ShapeDtypeStruct((B, 2), jnp.float32),
    scratch_types=[
        pltpu.VMEM((_BPW,), jnp.int32),
        pltpu.VMEM((_BPW, 2), jnp.float32),
    ],
    compiler_params=pltpu.CompilerParams(
        needs_layout_passes=False,
        use_tc_tiling_on_sc=True,
        skip_device_barrier=True,
    ),
)
def _router(it_hbm, out_hbm, it_v, out_v):
    wid = lax.axis_index("s") * _NUM_CORES + lax.axis_index("c")
    base = wid * _BPW
    pltpu.sync_copy(it_hbm.at[pl.ds(base, _BPW)], it_v)

    lane = lax.iota(jnp.int32, _LANES)
    parity = lane & 1
    half = lane >> 1

    for k in range(2 * _BPW // _LANES):
        row = half + (_LANES // 2) * k
        src = plsc.load_gather(it_v, [row])
        n = src + 1
        q = (n.astype(jnp.float32) * (1.0 / TAU) + 0.5).astype(jnp.int32)
        pred = (q * TAU == n).astype(jnp.int32)
        plsc.store_scatter(out_v, [row, parity], (pred == parity).astype(jnp.float32))

    pltpu.sync_copy(out_v, out_hbm.at[pl.ds(base, _BPW), :])


def kernel(iteration):
    return _router(iteration.astype(jnp.int32))


# speedup vs baseline: 3.1829x; 1.5975x over previous
import functools

import jax
import jax.numpy as jnp
from jax import lax
from jax.experimental import pallas as pl
from jax.experimental.pallas import tpu as pltpu
from jax.experimental.pallas import tpu_sc as plsc

B = 16384
TAU = 5
_NUM_CORES = 2
_NUM_SUBCORES = 16
_NW = _NUM_CORES * _NUM_SUBCORES
_BPW = B // _NW
_LANES = 16
_BLK_W = _BPW // 128

_mesh = plsc.VectorSubcoreMesh(core_axis_name="c", subcore_axis_name="s")


@functools.partial(
    pl.kernel,
    mesh=_mesh,
    out_type=jax.ShapeDtypeStruct((2 * B // 128, 128), jnp.float32),
    scratch_types=[
        pltpu.VMEM((_BPW,), jnp.int32),
        pltpu.VMEM((2 * _BLK_W, 128), jnp.float32),
    ],
    compiler_params=pltpu.CompilerParams(
        needs_layout_passes=False,
        skip_device_barrier=True,
    ),
)
def _router(it_hbm, out_hbm, it_v, out_v):
    wid = lax.axis_index("s") * _NUM_CORES + lax.axis_index("c")
    base = wid * _BPW
    pltpu.sync_copy(it_hbm.at[pl.ds(base, _BPW)], it_v)

    for blk in range(_BLK_W):
        for k in range(128 // _LANES):
            x = it_v[pl.ds(128 * blk + _LANES * k, _LANES)]
            n = x + 1
            q = (n.astype(jnp.float32) * (1.0 / TAU) + 0.5).astype(jnp.int32)
            hit = (q * TAU == n).astype(jnp.float32)
            out_v[2 * blk, pl.ds(_LANES * k, _LANES)] = 1.0 - hit
            out_v[2 * blk + 1, pl.ds(_LANES * k, _LANES)] = hit

    pltpu.sync_copy(out_v, out_hbm.at[pl.ds(wid * 2 * _BLK_W, 2 * _BLK_W), :])


def kernel(iteration):
    blocked = _router(iteration.astype(jnp.int32))
    return blocked.reshape(B // 128, 2, 128).transpose(0, 2, 1).reshape(B, 2)


# speedup vs baseline: 3.3226x; 1.0439x over previous
import functools

import jax
import jax.numpy as jnp
from jax import lax
from jax.experimental import pallas as pl
from jax.experimental.pallas import tpu as pltpu
from jax.experimental.pallas import tpu_sc as plsc

B = 16384
TAU = 5
_NUM_CORES = 1
_NUM_SUBCORES = 16
_NW = _NUM_CORES * _NUM_SUBCORES
_BPW = B // _NW
_LANES = 16
_BLK_W = _BPW // 128

_mesh = plsc.VectorSubcoreMesh(core_axis_name="c", subcore_axis_name="s", num_cores=1)


@functools.partial(
    pl.kernel,
    mesh=_mesh,
    out_type=jax.ShapeDtypeStruct((2 * B // 128, 128), jnp.float32),
    scratch_types=[
        pltpu.VMEM((_BPW,), jnp.int32),
        pltpu.VMEM((2 * _BLK_W, 128), jnp.float32),
    ],
    compiler_params=pltpu.CompilerParams(
        needs_layout_passes=False,
        skip_device_barrier=True,
    ),
)
def _router(it_hbm, out_hbm, it_v, out_v):
    wid = lax.axis_index("s") * _NUM_CORES + lax.axis_index("c")
    base = wid * _BPW
    pltpu.sync_copy(it_hbm.at[pl.ds(base, _BPW)], it_v)

    for blk in range(_BLK_W):
        for k in range(128 // _LANES):
            x = it_v[pl.ds(128 * blk + _LANES * k, _LANES)]
            n = x + 1
            q = (n.astype(jnp.float32) * (1.0 / TAU) + 0.5).astype(jnp.int32)
            hit = (q * TAU == n).astype(jnp.float32)
            out_v[2 * blk, pl.ds(_LANES * k, _LANES)] = 1.0 - hit
            out_v[2 * blk + 1, pl.ds(_LANES * k, _LANES)] = hit

    pltpu.sync_copy(out_v, out_hbm.at[pl.ds(wid * 2 * _BLK_W, 2 * _BLK_W), :])


def kernel(iteration):
    blocked = _router(iteration.astype(jnp.int32))
    return blocked.reshape(B // 128, 2, 128).transpose(0, 2, 1).reshape(B, 2)
